# baseline (speedup 1.0000x reference)
import functools

import jax
import jax.numpy as jnp
from jax import lax
from jax.experimental import pallas as pl
from jax.experimental.pallas import tpu as pltpu
from jax.experimental.pallas import tpu_sc as plsc

B = 16
N = 1024
KNN = 20
IMIN = -(2**31)

try:
    _info = plsc.get_sparse_core_info()
    _NC, _NS = _info.num_cores, _info.num_subcores
except ValueError:
    _NC, _NS = 2, 16
_NW = _NC * _NS


def _topk_body(xt_ref, xr_ref, idx_ref):
    b = pl.program_id(0)
    xt = xt_ref[0]
    xr = xr_ref[...]
    s = 2.0 * jnp.dot(xr, xt, preferred_element_type=jnp.float32)
    s = s - jnp.sum(xt * xt, axis=0, keepdims=True)
    u = lax.bitcast_convert_type(s, jnp.int32)
    col = lax.broadcasted_iota(jnp.int32, s.shape, 1)
    key = lax.bitcast_convert_type((u & -1024) | (1023 - col), jnp.float32)
    tops = [jnp.max(key, axis=1, keepdims=True)]
    for _ in range(KNN - 1):
        cand = jnp.where(key < tops[-1], key, -jnp.inf)
        tops.append(jnp.max(cand, axis=1, keepdims=True))
    m = lax.bitcast_convert_type(
        jnp.concatenate(tops, axis=1), jnp.int32)
    idx_ref[...] = (1023 - (m & 1023)) + b * N


def _topk(xt, a):
    nbat, c = xt.shape[0], a.shape[1]
    r = 256
    nb = N // r
    return pl.pallas_call(
        _topk_body,
        grid=(nbat, nb),
        in_specs=[
            pl.BlockSpec((1, c, N), lambda b, rb: (b, 0, 0)),
            pl.BlockSpec((r, c), lambda b, rb: (b * nb + rb, 0)),
        ],
        out_specs=pl.BlockSpec((r, KNN), lambda b, rb: (b * nb + rb, 0)),
        out_shape=jax.ShapeDtypeStruct((nbat * N, KNN), jnp.int32),
    )(xt, a)


def _lrelu(z):
    return jnp.where(z >= 0, z, 0.2 * z)


def _mm_uv_body(x_ref, aw_ref, dw_ref, b_ref, u_ref, v_ref):
    x = x_ref[...]
    u = jnp.dot(x, aw_ref[...], preferred_element_type=jnp.float32)
    u_ref[...] = u.astype(jnp.bfloat16)
    v = (jnp.dot(x, dw_ref[...], preferred_element_type=jnp.float32)
         + b_ref[...])
    v_ref[...] = v.astype(jnp.bfloat16)


def _mm_uv(a, aw, dw, bias):
    c, co = aw.shape
    cop = 256
    if cop != co:
        aw = jnp.concatenate(
            [aw, jnp.zeros((c, cop - co), jnp.float32)], axis=1)
    rows = a.shape[0]
    m = 512
    nb = rows // m
    return pl.pallas_call(
        _mm_uv_body,
        grid=(nb,),
        in_specs=[
            pl.BlockSpec((m, c), lambda i: (i, 0)),
            pl.BlockSpec((c, cop), lambda i: (0, 0)),
            pl.BlockSpec((c, co), lambda i: (0, 0)),
            pl.BlockSpec((1, co), lambda i: (0, 0)),
        ],
        out_specs=[
            pl.BlockSpec((m, cop), lambda i: (i, 0)),
            pl.BlockSpec((m, co), lambda i: (i, 0)),
        ],
        out_shape=[
            jax.ShapeDtypeStruct((rows, cop), jnp.bfloat16),
            jax.ShapeDtypeStruct((rows, co), jnp.bfloat16),
        ],
    )(a, aw, dw, bias.reshape(1, co))


def _mm_body(x_ref, w_ref, b_ref, o_ref, *, act):
    z = jnp.dot(x_ref[...], w_ref[...], preferred_element_type=jnp.float32)
    z = z + b_ref[...]
    o_ref[...] = _lrelu(z) if act else z


def _mm(x, w, bias, act=True, m=512):
    c, co = w.shape
    rows = x.shape[0]
    nb = rows // m
    return pl.pallas_call(
        functools.partial(_mm_body, act=act),
        grid=(nb,),
        in_specs=[
            pl.BlockSpec((m, c), lambda i: (i, 0)),
            pl.BlockSpec((c, co), lambda i: (0, 0)),
            pl.BlockSpec((1, co), lambda i: (0, 0)),
        ],
        out_specs=pl.BlockSpec((m, co), lambda i: (i, 0)),
        out_shape=jax.ShapeDtypeStruct((rows, co), jnp.float32),
    )(x, w, bias.reshape(1, co))


def _mm_rowadd_body(x_ref, w_ref, t_ref, o_ref):
    z = jnp.dot(x_ref[...], w_ref[...], preferred_element_type=jnp.float32)
    o_ref[...] = _lrelu(z + t_ref[0])


def _mm_rowadd(x, w, t):
    c, co = w.shape
    m = 512
    nb = (B * N) // m
    per_b = N // m
    return pl.pallas_call(
        _mm_rowadd_body,
        grid=(nb,),
        in_specs=[
            pl.BlockSpec((m, c), lambda i: (i, 0)),
            pl.BlockSpec((c, co), lambda i: (0, 0)),
            pl.BlockSpec((1, 1, co), lambda i: (i // per_b, 0, 0)),
        ],
        out_specs=pl.BlockSpec((m, co), lambda i: (i, 0)),
        out_shape=jax.ShapeDtypeStruct((B * N, co), jnp.float32),
    )(x, w, t)


def _pool_body(h_ref, mx_ref, av_ref):
    h = h_ref[...]
    mx_ref[0] = jnp.max(h, axis=0, keepdims=True)
    av_ref[0] = jnp.sum(h, axis=0, keepdims=True) * (1.0 / N)


def _pool(h):
    e = h.shape[1]
    return pl.pallas_call(
        _pool_body,
        grid=(B,),
        in_specs=[pl.BlockSpec((N, e), lambda b: (b, 0))],
        out_specs=[
            pl.BlockSpec((1, 1, e), lambda b: (b, 0, 0)),
            pl.BlockSpec((1, 1, e), lambda b: (b, 0, 0)),
        ],
        out_shape=[
            jax.ShapeDtypeStruct((B, 1, e), jnp.float32),
            jax.ShapeDtypeStruct((B, 1, e), jnp.float32),
        ],
    )(h)


def _tvec_body(g_ref, w_ref, b_ref, o_ref):
    o_ref[...] = (jnp.dot(g_ref[...], w_ref[...],
                          preferred_element_type=jnp.float32) + b_ref[...])


def _tvec(gcat, w, bias):
    c, co = w.shape
    return pl.pallas_call(
        _tvec_body,
        grid=(1,),
        in_specs=[
            pl.BlockSpec((B, c), lambda i: (0, 0)),
            pl.BlockSpec((c, co), lambda i: (0, 0)),
            pl.BlockSpec((1, co), lambda i: (0, 0)),
        ],
        out_specs=pl.BlockSpec((B, co), lambda i: (0, 0)),
        out_shape=jax.ShapeDtypeStruct((B, co), jnp.float32),
    )(gcat, w, bias.reshape(1, co))


def _mm_lsm_body(x_ref, w_ref, o_ref, *, valid):
    z = jnp.dot(x_ref[...], w_ref[...], preferred_element_type=jnp.float32)
    col = lax.broadcasted_iota(jnp.int32, z.shape, 1)
    ok = col < valid
    zm = jnp.where(ok, z, -jnp.inf)
    mx = jnp.max(zm, axis=1, keepdims=True)
    e = jnp.where(ok, jnp.exp(z - mx), 0.0)
    s = jnp.sum(e, axis=1, keepdims=True)
    o_ref[...] = z - mx - jnp.log(s)


def _mm_lsm(x, w, valid):
    c, co = w.shape
    m = 512
    nb = (B * N) // m
    return pl.pallas_call(
        functools.partial(_mm_lsm_body, valid=valid),
        grid=(nb,),
        in_specs=[
            pl.BlockSpec((m, c), lambda i: (i, 0)),
            pl.BlockSpec((c, co), lambda i: (0, 0)),
        ],
        out_specs=pl.BlockSpec((m, co), lambda i: (i, 0)),
        out_shape=jax.ShapeDtypeStruct((B * N, co), jnp.float32),
    )(x, w)


def _sc_gather_max(u, v, idx, co):
    pts = u.shape[0]
    u32 = lax.bitcast_convert_type(
        u.reshape(pts, u.shape[1] // 2, 2), jnp.int32)
    v32 = lax.bitcast_convert_type(
        v.reshape(pts, co // 2, 2), jnp.int32)
    cw = co // 2
    per_w = pts // _NW
    ch = 16 if co <= 128 else 8
    rounds = per_w // ch
    mesh = plsc.VectorSubcoreMesh(
        core_axis_name="c", subcore_axis_name="s",
        num_cores=_NC, num_subcores=_NS)

    @functools.partial(
        pl.kernel,
        out_type=jax.ShapeDtypeStruct((pts, cw), jnp.int32),
        compiler_params=pltpu.CompilerParams(needs_layout_passes=False),
        mesh=mesh,
        scratch_types=[
            pltpu.VMEM((ch * KNN,), jnp.int32),
            pltpu.VMEM((ch * KNN,), jnp.int32),
            pltpu.VMEM((ch * KNN, 128), jnp.int32),
            pltpu.VMEM((ch * KNN, 128), jnp.int32),
            pltpu.VMEM((ch, cw), jnp.int32),
            pltpu.VMEM((ch, cw), jnp.int32),
            pltpu.SemaphoreType.DMA,
            pltpu.SemaphoreType.DMA,
        ],
    )
    def k(u_hbm, v_hbm, idx_hbm, out_hbm,
          idx0, idx1, rows0, rows1, v_v, o_v, sem0, sem1):
        wid = lax.axis_index("s") * _NC + lax.axis_index("c")
        base = wid * per_w
        idx_b = (idx0, idx1)
        rows_b = (rows0, rows1)
        sems = (sem0, sem1)

        def issue(r, b):
            p0 = base + r * ch
            pltpu.sync_copy(idx_hbm.at[pl.ds(p0 * KNN, ch * KNN)], idx_b[b])
            pltpu.async_copy(u_hbm.at[idx_b[b]], rows_b[b], sems[b])

        issue(0, 0)

        def pair_body(i, carry):
            for b in range(2):
                r = 2 * i + b
                nxt = r + 1

                @pl.when(nxt < rounds)
                def _():
                    issue(nxt, 1 - b)

                pltpu.make_async_copy(
                    u_hbm.at[idx_b[b]], rows_b[b], sems[b]).wait()
                p0 = base + r * ch
                pltpu.sync_copy(v_hbm.at[pl.ds(p0, ch)], v_v)
                rows = rows_b[b]

                bf = jnp.bfloat16
                for p in range(ch):
                    for c in range(cw // 16):
                        sl = pl.ds(c * 16, 16)
                        vals = [plsc.bitcast(rows[p * KNN + q, sl], bf)
                                for q in range(KNN)]
                        while len(vals) > 1:
                            nxt_vals = []
                            for j in range(0, len(vals) - 1, 2):
                                nxt_vals.append(
                                    jnp.maximum(vals[j], vals[j + 1]))
                            if len(vals) % 2:
                                nxt_vals.append(vals[-1])
                            vals = nxt_vals
                        z = vals[0] + plsc.bitcast(v_v[p, sl], bf)
                        z = jnp.where(z >= 0, z, 0.2 * z)
                        o_v[p, sl] = plsc.bitcast(z, jnp.int32)
                pltpu.sync_copy(o_v, out_hbm.at[pl.ds(p0, ch)])
            return carry

        lax.fori_loop(0, rounds // 2, pair_body, 0)

    out32 = k(u32, v32, idx.reshape(-1))
    out = lax.bitcast_convert_type(out32, jnp.bfloat16)
    return out.reshape(pts, co).astype(jnp.float32)


def _edge_stage(a, w, g, bias):
    c = a.shape[1]
    co = w.shape[0]
    sg = (g / jnp.sqrt(1.0 + 1e-5))[:, None]
    aw = (w[:, :c] * sg).T
    dw = ((w[:, c:] - w[:, :c]) * sg).T
    nbat = a.shape[0] // N
    xt = a.reshape(nbat, N, c).transpose(0, 2, 1)
    idx = _topk(xt, a)
    u, v = _mm_uv(a, aw, dw, bias)
    return _sc_gather_max(u, v, idx, co)


def kernel(x, params):
    p = params
    a0 = x.transpose(0, 2, 1).reshape(B * N, 3)

    nchain = 2
    part = (B // nchain) * N
    feats = []
    for h in range(nchain):
        a = a0[h * part:(h + 1) * part]
        x1 = _edge_stage(a, p['W1'], p['g1'], p['b1'])
        x2 = _edge_stage(x1, p['W2'], p['g2'], p['b2'])
        x3 = _edge_stage(x2, p['W3'], p['g3'], p['b3'])
        x4 = _edge_stage(x3, p['W4'], p['g4'], p['b4'])
        feats.append((x1, x2, x3, x4))

    x1, x2, x3, x4 = (
        jnp.concatenate([f[i] for f in feats], axis=0) for i in range(4))
    cat4 = jnp.concatenate([x1, x2, x3, x4], axis=1)
    s5 = (p['g5'] / jnp.sqrt(1.0 + 1e-3))[:, None]
    h5 = _mm(cat4, (p['W5'] * s5).T, p['b5'])

    gmax, gavg = _pool(h5)
    gcat = jnp.concatenate([gmax[:, 0], gavg[:, 0]], axis=1)

    emb2 = gcat.shape[1]
    s106 = (p['g108'] / jnp.sqrt(1.0 + 1e-3))[:, None]
    w106 = p['W106'] * s106
    t = _tvec(gcat, w106[:, :emb2].T, p['b108'])
    h6 = _mm_rowadd(x2, w106[:, emb2:].T, t[:, None, :])

    s107 = (p['g109'] / jnp.sqrt(1.0 + 1e-3))[:, None]
    h7 = _mm(h6, (p['W107'] * s107).T, p['b109'])
    s108 = (p['g1010'] / jnp.sqrt(1.0 + 1e-3))[:, None]
    h8 = _mm(h7, (p['W108'] * s108).T, p['b1010'])

    w109 = jnp.zeros((128, 32), jnp.float32).at[:, :27].set(p['W109'].T)
    out = _mm_lsm(h8, w109, 27)
    return out.reshape(B, N, 32)[:, :, :27]


# speedup vs baseline: 1.6873x; 1.6873x over previous
import functools

import jax
import jax.numpy as jnp
from jax import lax
from jax.experimental import pallas as pl
from jax.experimental.pallas import tpu as pltpu
from jax.experimental.pallas import tpu_sc as plsc

B = 16
N = 1024
KNN = 20
IMIN = -(2**31)

try:
    _info = plsc.get_sparse_core_info()
    _NC, _NS = _info.num_cores, _info.num_subcores
except ValueError:
    _NC, _NS = 2, 16
_NW = _NC * _NS


def _topk_body(xt_ref, xr_ref, idx_ref):
    b = pl.program_id(0)
    xt = xt_ref[0]
    xr = xr_ref[...]
    s = 2.0 * jnp.dot(xr, xt, preferred_element_type=jnp.float32)
    s = s - jnp.sum(xt * xt, axis=0, keepdims=True)
    u = lax.bitcast_convert_type(s, jnp.int32)
    col = lax.broadcasted_iota(jnp.int32, s.shape, 1)
    key = lax.bitcast_convert_type((u & -1024) | (1023 - col), jnp.float32)
    tops = [jnp.max(key, axis=1, keepdims=True)]
    for _ in range(KNN - 1):
        cand = jnp.where(key < tops[-1], key, -jnp.inf)
        tops.append(jnp.max(cand, axis=1, keepdims=True))
    m = lax.bitcast_convert_type(
        jnp.concatenate(tops, axis=1), jnp.int32)
    idx_ref[...] = (1023 - (m & 1023)) + b * N


def _topk(xt, a):
    nbat, c = xt.shape[0], a.shape[1]
    r = 256
    nb = N // r
    return pl.pallas_call(
        _topk_body,
        grid=(nbat, nb),
        in_specs=[
            pl.BlockSpec((1, c, N), lambda b, rb: (b, 0, 0)),
            pl.BlockSpec((r, c), lambda b, rb: (b * nb + rb, 0)),
        ],
        out_specs=pl.BlockSpec((r, KNN), lambda b, rb: (b * nb + rb, 0)),
        out_shape=jax.ShapeDtypeStruct((nbat * N, KNN), jnp.int32),
    )(xt, a)


def _lrelu(z):
    return jnp.where(z >= 0, z, 0.2 * z)


def _mm_uv_body(x_ref, aw_ref, dw_ref, b_ref, u_ref, v_ref):
    x = x_ref[...]
    u_ref[...] = jnp.dot(x, aw_ref[...], preferred_element_type=jnp.float32)
    v_ref[...] = (jnp.dot(x, dw_ref[...], preferred_element_type=jnp.float32)
                  + b_ref[...])


def _mm_uv(a, aw, dw, bias):
    c, co = aw.shape
    cop = max(co, 128)
    if cop != co:
        aw = jnp.concatenate(
            [aw, jnp.zeros((c, cop - co), jnp.float32)], axis=1)
    rows = a.shape[0]
    m = 512
    nb = rows // m
    return pl.pallas_call(
        _mm_uv_body,
        grid=(nb,),
        in_specs=[
            pl.BlockSpec((m, c), lambda i: (i, 0)),
            pl.BlockSpec((c, cop), lambda i: (0, 0)),
            pl.BlockSpec((c, co), lambda i: (0, 0)),
            pl.BlockSpec((1, co), lambda i: (0, 0)),
        ],
        out_specs=[
            pl.BlockSpec((m, cop), lambda i: (i, 0)),
            pl.BlockSpec((m, co), lambda i: (i, 0)),
        ],
        out_shape=[
            jax.ShapeDtypeStruct((rows, cop), jnp.float32),
            jax.ShapeDtypeStruct((rows, co), jnp.float32),
        ],
    )(a, aw, dw, bias.reshape(1, co))


def _mm_body(x_ref, w_ref, b_ref, o_ref, *, act):
    z = jnp.dot(x_ref[...], w_ref[...], preferred_element_type=jnp.float32)
    z = z + b_ref[...]
    o_ref[...] = _lrelu(z) if act else z


def _mm(x, w, bias, act=True, m=512):
    c, co = w.shape
    rows = x.shape[0]
    nb = rows // m
    return pl.pallas_call(
        functools.partial(_mm_body, act=act),
        grid=(nb,),
        in_specs=[
            pl.BlockSpec((m, c), lambda i: (i, 0)),
            pl.BlockSpec((c, co), lambda i: (0, 0)),
            pl.BlockSpec((1, co), lambda i: (0, 0)),
        ],
        out_specs=pl.BlockSpec((m, co), lambda i: (i, 0)),
        out_shape=jax.ShapeDtypeStruct((rows, co), jnp.float32),
    )(x, w, bias.reshape(1, co))


def _mm_rowadd_body(x_ref, w_ref, t_ref, o_ref):
    z = jnp.dot(x_ref[...], w_ref[...], preferred_element_type=jnp.float32)
    o_ref[...] = _lrelu(z + t_ref[0])


def _mm_rowadd(x, w, t):
    c, co = w.shape
    m = 512
    nb = (B * N) // m
    per_b = N // m
    return pl.pallas_call(
        _mm_rowadd_body,
        grid=(nb,),
        in_specs=[
            pl.BlockSpec((m, c), lambda i: (i, 0)),
            pl.BlockSpec((c, co), lambda i: (0, 0)),
            pl.BlockSpec((1, 1, co), lambda i: (i // per_b, 0, 0)),
        ],
        out_specs=pl.BlockSpec((m, co), lambda i: (i, 0)),
        out_shape=jax.ShapeDtypeStruct((B * N, co), jnp.float32),
    )(x, w, t)


def _pool_body(h_ref, mx_ref, av_ref):
    h = h_ref[...]
    mx_ref[0] = jnp.max(h, axis=0, keepdims=True)
    av_ref[0] = jnp.sum(h, axis=0, keepdims=True) * (1.0 / N)


def _pool(h):
    e = h.shape[1]
    return pl.pallas_call(
        _pool_body,
        grid=(B,),
        in_specs=[pl.BlockSpec((N, e), lambda b: (b, 0))],
        out_specs=[
            pl.BlockSpec((1, 1, e), lambda b: (b, 0, 0)),
            pl.BlockSpec((1, 1, e), lambda b: (b, 0, 0)),
        ],
        out_shape=[
            jax.ShapeDtypeStruct((B, 1, e), jnp.float32),
            jax.ShapeDtypeStruct((B, 1, e), jnp.float32),
        ],
    )(h)


def _tvec_body(g_ref, w_ref, b_ref, o_ref):
    o_ref[...] = (jnp.dot(g_ref[...], w_ref[...],
                          preferred_element_type=jnp.float32) + b_ref[...])


def _tvec(gcat, w, bias):
    c, co = w.shape
    return pl.pallas_call(
        _tvec_body,
        grid=(1,),
        in_specs=[
            pl.BlockSpec((B, c), lambda i: (0, 0)),
            pl.BlockSpec((c, co), lambda i: (0, 0)),
            pl.BlockSpec((1, co), lambda i: (0, 0)),
        ],
        out_specs=pl.BlockSpec((B, co), lambda i: (0, 0)),
        out_shape=jax.ShapeDtypeStruct((B, co), jnp.float32),
    )(gcat, w, bias.reshape(1, co))


def _mm_lsm_body(x_ref, w_ref, o_ref, *, valid):
    z = jnp.dot(x_ref[...], w_ref[...], preferred_element_type=jnp.float32)
    col = lax.broadcasted_iota(jnp.int32, z.shape, 1)
    ok = col < valid
    zm = jnp.where(ok, z, -jnp.inf)
    mx = jnp.max(zm, axis=1, keepdims=True)
    e = jnp.where(ok, jnp.exp(z - mx), 0.0)
    s = jnp.sum(e, axis=1, keepdims=True)
    o_ref[...] = z - mx - jnp.log(s)


def _mm_lsm(x, w, valid):
    c, co = w.shape
    m = 512
    nb = (B * N) // m
    return pl.pallas_call(
        functools.partial(_mm_lsm_body, valid=valid),
        grid=(nb,),
        in_specs=[
            pl.BlockSpec((m, c), lambda i: (i, 0)),
            pl.BlockSpec((c, co), lambda i: (0, 0)),
        ],
        out_specs=pl.BlockSpec((m, co), lambda i: (i, 0)),
        out_shape=jax.ShapeDtypeStruct((B * N, co), jnp.float32),
    )(x, w)


def _sc_gather_max(u, v, idx, co):
    pts = u.shape[0]
    cop = u.shape[1]
    per_w = pts // _NW
    ch = 16 if cop <= 128 else 8
    rounds = per_w // ch
    mesh = plsc.VectorSubcoreMesh(
        core_axis_name="c", subcore_axis_name="s",
        num_cores=_NC, num_subcores=_NS)

    @functools.partial(
        pl.kernel,
        out_type=jax.ShapeDtypeStruct((pts, co), jnp.float32),
        mesh=mesh,
        scratch_types=[
            pltpu.VMEM((ch * KNN,), jnp.int32),
            pltpu.VMEM((ch * KNN,), jnp.int32),
            pltpu.VMEM((ch * KNN, cop), jnp.float32),
            pltpu.VMEM((ch * KNN, cop), jnp.float32),
            pltpu.VMEM((ch, co), jnp.float32),
            pltpu.VMEM((ch, co), jnp.float32),
            pltpu.SemaphoreType.DMA,
            pltpu.SemaphoreType.DMA,
        ],
    )
    def k(u_hbm, v_hbm, idx_hbm, out_hbm,
          idx0, idx1, rows0, rows1, v_v, o_v, sem0, sem1):
        wid = lax.axis_index("s") * _NC + lax.axis_index("c")
        base = wid * per_w
        idx_b = (idx0, idx1)
        rows_b = (rows0, rows1)
        sems = (sem0, sem1)

        def issue(r, b):
            p0 = base + r * ch
            pltpu.sync_copy(idx_hbm.at[pl.ds(p0 * KNN, ch * KNN)], idx_b[b])
            pltpu.async_copy(u_hbm.at[idx_b[b]], rows_b[b], sems[b])

        issue(0, 0)

        def pair_body(i, carry):
            for b in range(2):
                r = 2 * i + b
                nxt = r + 1

                @pl.when(nxt < rounds)
                def _():
                    issue(nxt, 1 - b)

                pltpu.make_async_copy(
                    u_hbm.at[idx_b[b]], rows_b[b], sems[b]).wait()
                p0 = base + r * ch
                pltpu.sync_copy(v_hbm.at[pl.ds(p0, ch)], v_v)
                rows = rows_b[b]

                def pt_body(p, c2):
                    for c in range(co // 16):
                        sl = pl.ds(c * 16, 16)
                        vals = [rows[p * KNN + q, sl] for q in range(KNN)]
                        while len(vals) > 1:
                            nxt_vals = []
                            for j in range(0, len(vals) - 1, 2):
                                nxt_vals.append(
                                    jnp.maximum(vals[j], vals[j + 1]))
                            if len(vals) % 2:
                                nxt_vals.append(vals[-1])
                            vals = nxt_vals
                        z = vals[0] + v_v[p, sl]
                        o_v[p, sl] = jnp.where(z >= 0, z, 0.2 * z)
                    return c2

                lax.fori_loop(0, ch, pt_body, 0)
                pltpu.sync_copy(o_v, out_hbm.at[pl.ds(p0, ch)])
            return carry

        lax.fori_loop(0, rounds // 2, pair_body, 0)

    return k(u, v, idx.reshape(-1))


def _edge_stage(a, w, g, bias):
    c = a.shape[1]
    co = w.shape[0]
    sg = (g / jnp.sqrt(1.0 + 1e-5))[:, None]
    aw = (w[:, :c] * sg).T
    dw = ((w[:, c:] - w[:, :c]) * sg).T
    nbat = a.shape[0] // N
    xt = a.reshape(nbat, N, c).transpose(0, 2, 1)
    idx = _topk(xt, a)
    u, v = _mm_uv(a, aw, dw, bias)
    return _sc_gather_max(u, v, idx, co)


def kernel(x, params):
    p = params
    a0 = x.transpose(0, 2, 1).reshape(B * N, 3)

    nchain = 2
    part = (B // nchain) * N
    feats = []
    for h in range(nchain):
        a = a0[h * part:(h + 1) * part]
        x1 = _edge_stage(a, p['W1'], p['g1'], p['b1'])
        x2 = _edge_stage(x1, p['W2'], p['g2'], p['b2'])
        x3 = _edge_stage(x2, p['W3'], p['g3'], p['b3'])
        x4 = _edge_stage(x3, p['W4'], p['g4'], p['b4'])
        feats.append((x1, x2, x3, x4))

    x1, x2, x3, x4 = (
        jnp.concatenate([f[i] for f in feats], axis=0) for i in range(4))
    cat4 = jnp.concatenate([x1, x2, x3, x4], axis=1)
    s5 = (p['g5'] / jnp.sqrt(1.0 + 1e-3))[:, None]
    h5 = _mm(cat4, (p['W5'] * s5).T, p['b5'])

    gmax, gavg = _pool(h5)
    gcat = jnp.concatenate([gmax[:, 0], gavg[:, 0]], axis=1)

    emb2 = gcat.shape[1]
    s106 = (p['g108'] / jnp.sqrt(1.0 + 1e-3))[:, None]
    w106 = p['W106'] * s106
    t = _tvec(gcat, w106[:, :emb2].T, p['b108'])
    h6 = _mm_rowadd(x2, w106[:, emb2:].T, t[:, None, :])

    s107 = (p['g109'] / jnp.sqrt(1.0 + 1e-3))[:, None]
    h7 = _mm(h6, (p['W107'] * s107).T, p['b109'])
    s108 = (p['g1010'] / jnp.sqrt(1.0 + 1e-3))[:, None]
    h8 = _mm(h7, (p['W108'] * s108).T, p['b1010'])

    w109 = jnp.zeros((128, 32), jnp.float32).at[:, :27].set(p['W109'].T)
    out = _mm_lsm(h8, w109, 27)
    return out.reshape(B, N, 32)[:, :, :27]


# speedup vs baseline: 1.7773x; 1.0534x over previous
import functools

import jax
import jax.numpy as jnp
from jax import lax
from jax.experimental import pallas as pl
from jax.experimental.pallas import tpu as pltpu
from jax.experimental.pallas import tpu_sc as plsc

B = 16
N = 1024
KNN = 20
IMIN = -(2**31)

try:
    _info = plsc.get_sparse_core_info()
    _NC, _NS = _info.num_cores, _info.num_subcores
except ValueError:
    _NC, _NS = 2, 16
_NW = _NC * _NS


def _topk_body(xt_ref, xr_ref, idx_ref):
    b = pl.program_id(0)
    xt = xt_ref[0]
    xr = xr_ref[...]
    s = 2.0 * jnp.dot(xr, xt, preferred_element_type=jnp.float32)
    s = s - jnp.sum(xt * xt, axis=0, keepdims=True)
    u = lax.bitcast_convert_type(s, jnp.int32)
    col = lax.broadcasted_iota(jnp.int32, s.shape, 1)
    key = lax.bitcast_convert_type((u & -1024) | (1023 - col), jnp.float32)
    tops = [jnp.max(key, axis=1, keepdims=True)]
    for _ in range(KNN - 1):
        cand = jnp.where(key < tops[-1], key, -jnp.inf)
        tops.append(jnp.max(cand, axis=1, keepdims=True))
    m = lax.bitcast_convert_type(
        jnp.concatenate(tops, axis=1), jnp.int32)
    idx_ref[...] = (1023 - (m & 1023)) + b * N


def _topk(xt, a):
    nbat, c = xt.shape[0], a.shape[1]
    r = 256
    nb = N // r
    return pl.pallas_call(
        _topk_body,
        grid=(nbat, nb),
        in_specs=[
            pl.BlockSpec((1, c, N), lambda b, rb: (b, 0, 0)),
            pl.BlockSpec((r, c), lambda b, rb: (b * nb + rb, 0)),
        ],
        out_specs=pl.BlockSpec((r, KNN), lambda b, rb: (b * nb + rb, 0)),
        out_shape=jax.ShapeDtypeStruct((nbat * N, KNN), jnp.int32),
    )(xt, a)


def _lrelu(z):
    return jnp.where(z >= 0, z, 0.2 * z)


def _mm_uv_body(x_ref, aw_ref, dw_ref, b_ref, u_ref, v_ref):
    x = x_ref[...]
    u_ref[...] = jnp.dot(x, aw_ref[...], preferred_element_type=jnp.float32)
    v_ref[...] = (jnp.dot(x, dw_ref[...], preferred_element_type=jnp.float32)
                  + b_ref[...])


def _mm_uv(a, aw, dw, bias):
    c, co = aw.shape
    cop = max(co, 128)
    if cop != co:
        aw = jnp.concatenate(
            [aw, jnp.zeros((c, cop - co), jnp.float32)], axis=1)
    rows = a.shape[0]
    m = 512
    nb = rows // m
    return pl.pallas_call(
        _mm_uv_body,
        grid=(nb,),
        in_specs=[
            pl.BlockSpec((m, c), lambda i: (i, 0)),
            pl.BlockSpec((c, cop), lambda i: (0, 0)),
            pl.BlockSpec((c, co), lambda i: (0, 0)),
            pl.BlockSpec((1, co), lambda i: (0, 0)),
        ],
        out_specs=[
            pl.BlockSpec((m, cop), lambda i: (i, 0)),
            pl.BlockSpec((m, co), lambda i: (i, 0)),
        ],
        out_shape=[
            jax.ShapeDtypeStruct((rows, cop), jnp.float32),
            jax.ShapeDtypeStruct((rows, co), jnp.float32),
        ],
    )(a, aw, dw, bias.reshape(1, co))


def _mm_body(x_ref, w_ref, b_ref, o_ref, *, act):
    z = jnp.dot(x_ref[...], w_ref[...], preferred_element_type=jnp.float32)
    z = z + b_ref[...]
    o_ref[...] = _lrelu(z) if act else z


def _mm(x, w, bias, act=True, m=512):
    c, co = w.shape
    rows = x.shape[0]
    nb = rows // m
    return pl.pallas_call(
        functools.partial(_mm_body, act=act),
        grid=(nb,),
        in_specs=[
            pl.BlockSpec((m, c), lambda i: (i, 0)),
            pl.BlockSpec((c, co), lambda i: (0, 0)),
            pl.BlockSpec((1, co), lambda i: (0, 0)),
        ],
        out_specs=pl.BlockSpec((m, co), lambda i: (i, 0)),
        out_shape=jax.ShapeDtypeStruct((rows, co), jnp.float32),
    )(x, w, bias.reshape(1, co))


def _mm_rowadd_body(x_ref, w_ref, t_ref, o_ref):
    z = jnp.dot(x_ref[...], w_ref[...], preferred_element_type=jnp.float32)
    o_ref[...] = _lrelu(z + t_ref[0])


def _mm_rowadd(x, w, t):
    c, co = w.shape
    m = 512
    nb = (B * N) // m
    per_b = N // m
    return pl.pallas_call(
        _mm_rowadd_body,
        grid=(nb,),
        in_specs=[
            pl.BlockSpec((m, c), lambda i: (i, 0)),
            pl.BlockSpec((c, co), lambda i: (0, 0)),
            pl.BlockSpec((1, 1, co), lambda i: (i // per_b, 0, 0)),
        ],
        out_specs=pl.BlockSpec((m, co), lambda i: (i, 0)),
        out_shape=jax.ShapeDtypeStruct((B * N, co), jnp.float32),
    )(x, w, t)


def _pool_body(h_ref, mx_ref, av_ref):
    h = h_ref[...]
    mx_ref[0] = jnp.max(h, axis=0, keepdims=True)
    av_ref[0] = jnp.sum(h, axis=0, keepdims=True) * (1.0 / N)


def _pool(h):
    e = h.shape[1]
    return pl.pallas_call(
        _pool_body,
        grid=(B,),
        in_specs=[pl.BlockSpec((N, e), lambda b: (b, 0))],
        out_specs=[
            pl.BlockSpec((1, 1, e), lambda b: (b, 0, 0)),
            pl.BlockSpec((1, 1, e), lambda b: (b, 0, 0)),
        ],
        out_shape=[
            jax.ShapeDtypeStruct((B, 1, e), jnp.float32),
            jax.ShapeDtypeStruct((B, 1, e), jnp.float32),
        ],
    )(h)


def _tvec_body(g_ref, w_ref, b_ref, o_ref):
    o_ref[...] = (jnp.dot(g_ref[...], w_ref[...],
                          preferred_element_type=jnp.float32) + b_ref[...])


def _tvec(gcat, w, bias):
    c, co = w.shape
    return pl.pallas_call(
        _tvec_body,
        grid=(1,),
        in_specs=[
            pl.BlockSpec((B, c), lambda i: (0, 0)),
            pl.BlockSpec((c, co), lambda i: (0, 0)),
            pl.BlockSpec((1, co), lambda i: (0, 0)),
        ],
        out_specs=pl.BlockSpec((B, co), lambda i: (0, 0)),
        out_shape=jax.ShapeDtypeStruct((B, co), jnp.float32),
    )(gcat, w, bias.reshape(1, co))


def _mm_lsm_body(x_ref, w_ref, o_ref, *, valid):
    z = jnp.dot(x_ref[...], w_ref[...], preferred_element_type=jnp.float32)
    col = lax.broadcasted_iota(jnp.int32, z.shape, 1)
    ok = col < valid
    zm = jnp.where(ok, z, -jnp.inf)
    mx = jnp.max(zm, axis=1, keepdims=True)
    e = jnp.where(ok, jnp.exp(z - mx), 0.0)
    s = jnp.sum(e, axis=1, keepdims=True)
    o_ref[...] = z - mx - jnp.log(s)


def _mm_lsm(x, w, valid):
    c, co = w.shape
    m = 512
    nb = (B * N) // m
    return pl.pallas_call(
        functools.partial(_mm_lsm_body, valid=valid),
        grid=(nb,),
        in_specs=[
            pl.BlockSpec((m, c), lambda i: (i, 0)),
            pl.BlockSpec((c, co), lambda i: (0, 0)),
        ],
        out_specs=pl.BlockSpec((m, co), lambda i: (i, 0)),
        out_shape=jax.ShapeDtypeStruct((B * N, co), jnp.float32),
    )(x, w)


def _sc_gather_max(u, v, idx, co):
    pts = u.shape[0]
    cop = u.shape[1]
    per_w = pts // _NW
    ch = 16 if cop <= 128 else 8
    rounds = per_w // ch
    mesh = plsc.VectorSubcoreMesh(
        core_axis_name="c", subcore_axis_name="s",
        num_cores=_NC, num_subcores=_NS)

    @functools.partial(
        pl.kernel,
        out_type=jax.ShapeDtypeStruct((pts, co), jnp.float32),
        mesh=mesh,
        scratch_types=[
            pltpu.VMEM((ch * KNN,), jnp.int32),
            pltpu.VMEM((ch * KNN,), jnp.int32),
            pltpu.VMEM((ch * KNN, cop), jnp.float32),
            pltpu.VMEM((ch * KNN, cop), jnp.float32),
            pltpu.VMEM((ch, co), jnp.float32),
            pltpu.VMEM((ch, co), jnp.float32),
            pltpu.VMEM((ch, co), jnp.float32),
            pltpu.VMEM((ch, co), jnp.float32),
            pltpu.SemaphoreType.DMA,
            pltpu.SemaphoreType.DMA,
            pltpu.SemaphoreType.DMA,
            pltpu.SemaphoreType.DMA,
            pltpu.SemaphoreType.DMA,
            pltpu.SemaphoreType.DMA,
        ],
    )
    def k(u_hbm, v_hbm, idx_hbm, out_hbm,
          idx0, idx1, rows0, rows1, v0, v1, o0, o1,
          gs0, gs1, vs0, vs1, os0, os1):
        wid = lax.axis_index("s") * _NC + lax.axis_index("c")
        base = wid * per_w
        idx_b = (idx0, idx1)
        rows_b = (rows0, rows1)
        v_b = (v0, v1)
        o_b = (o0, o1)
        gsems = (gs0, gs1)
        vsems = (vs0, vs1)
        osems = (os0, os1)

        def issue(r, b):
            p0 = base + r * ch
            pltpu.sync_copy(idx_hbm.at[pl.ds(p0 * KNN, ch * KNN)], idx_b[b])
            pltpu.async_copy(u_hbm.at[idx_b[b]], rows_b[b], gsems[b])
            pltpu.async_copy(v_hbm.at[pl.ds(p0, ch)], v_b[b], vsems[b])

        issue(0, 0)

        def pair_body(i, carry):
            for b in range(2):
                r = 2 * i + b
                nxt = r + 1

                @pl.when(nxt < rounds)
                def _():
                    issue(nxt, 1 - b)

                p0 = base + r * ch

                @pl.when(r >= 2)
                def _():
                    pltpu.make_async_copy(
                        o_b[b], out_hbm.at[pl.ds(p0, ch)], osems[b]).wait()

                pltpu.make_async_copy(
                    u_hbm.at[idx_b[b]], rows_b[b], gsems[b]).wait()
                pltpu.make_async_copy(
                    v_hbm.at[pl.ds(p0, ch)], v_b[b], vsems[b]).wait()
                rows = rows_b[b]
                v_v = v_b[b]
                o_v = o_b[b]

                def pt_body(p, c2):
                    for c in range(co // 16):
                        sl = pl.ds(c * 16, 16)
                        vals = [rows[p * KNN + q, sl] for q in range(KNN)]
                        while len(vals) > 1:
                            nxt_vals = []
                            for j in range(0, len(vals) - 1, 2):
                                nxt_vals.append(
                                    jnp.maximum(vals[j], vals[j + 1]))
                            if len(vals) % 2:
                                nxt_vals.append(vals[-1])
                            vals = nxt_vals
                        z = vals[0] + v_v[p, sl]
                        o_v[p, sl] = jnp.where(z >= 0, z, 0.2 * z)
                    return c2

                lax.fori_loop(0, ch, pt_body, 0)
                pltpu.async_copy(o_v, out_hbm.at[pl.ds(p0, ch)], osems[b])
            return carry

        lax.fori_loop(0, rounds // 2, pair_body, 0)
        for b in range(2):
            r = rounds - 2 + b
            p0 = base + r * ch
            pltpu.make_async_copy(
                o_b[b], out_hbm.at[pl.ds(p0, ch)], osems[b]).wait()

    return k(u, v, idx.reshape(-1))


def _edge_stage(a, w, g, bias):
    c = a.shape[1]
    co = w.shape[0]
    sg = (g / jnp.sqrt(1.0 + 1e-5))[:, None]
    aw = (w[:, :c] * sg).T
    dw = ((w[:, c:] - w[:, :c]) * sg).T
    nbat = a.shape[0] // N
    xt = a.reshape(nbat, N, c).transpose(0, 2, 1)
    idx = _topk(xt, a)
    u, v = _mm_uv(a, aw, dw, bias)
    return _sc_gather_max(u, v, idx, co)


def kernel(x, params):
    p = params
    a0 = x.transpose(0, 2, 1).reshape(B * N, 3)

    nchain = 2
    part = (B // nchain) * N
    feats = []
    for h in range(nchain):
        a = a0[h * part:(h + 1) * part]
        x1 = _edge_stage(a, p['W1'], p['g1'], p['b1'])
        x2 = _edge_stage(x1, p['W2'], p['g2'], p['b2'])
        x3 = _edge_stage(x2, p['W3'], p['g3'], p['b3'])
        x4 = _edge_stage(x3, p['W4'], p['g4'], p['b4'])
        feats.append((x1, x2, x3, x4))

    x1, x2, x3, x4 = (
        jnp.concatenate([f[i] for f in feats], axis=0) for i in range(4))
    cat4 = jnp.concatenate([x1, x2, x3, x4], axis=1)
    s5 = (p['g5'] / jnp.sqrt(1.0 + 1e-3))[:, None]
    h5 = _mm(cat4, (p['W5'] * s5).T, p['b5'])

    gmax, gavg = _pool(h5)
    gcat = jnp.concatenate([gmax[:, 0], gavg[:, 0]], axis=1)

    emb2 = gcat.shape[1]
    s106 = (p['g108'] / jnp.sqrt(1.0 + 1e-3))[:, None]
    w106 = p['W106'] * s106
    t = _tvec(gcat, w106[:, :emb2].T, p['b108'])
    h6 = _mm_rowadd(x2, w106[:, emb2:].T, t[:, None, :])

    s107 = (p['g109'] / jnp.sqrt(1.0 + 1e-3))[:, None]
    h7 = _mm(h6, (p['W107'] * s107).T, p['b109'])
    s108 = (p['g1010'] / jnp.sqrt(1.0 + 1e-3))[:, None]
    h8 = _mm(h7, (p['W108'] * s108).T, p['b1010'])

    w109 = jnp.zeros((128, 32), jnp.float32).at[:, :27].set(p['W109'].T)
    out = _mm_lsm(h8, w109, 27)
    return out.reshape(B, N, 32)[:, :, :27]


# speedup vs baseline: 1.8216x; 1.0250x over previous
import functools

import jax
import jax.numpy as jnp
from jax import lax
from jax.experimental import pallas as pl
from jax.experimental.pallas import tpu as pltpu
from jax.experimental.pallas import tpu_sc as plsc

B = 16
N = 1024
KNN = 20
IMIN = -(2**31)

try:
    _info = plsc.get_sparse_core_info()
    _NC, _NS = _info.num_cores, _info.num_subcores
except ValueError:
    _NC, _NS = 2, 16
_NW = _NC * _NS


def _topk_body(xb_ref, xr_ref, idx_ref):
    b = pl.program_id(0)
    xb = xb_ref[...]
    xr = xr_ref[...]
    dn = (((1,), (1,)), ((), ()))
    s = 2.0 * lax.dot_general(xr, xb, dn,
                              preferred_element_type=jnp.float32)
    csq = lax.dot_general(jnp.ones((8, xb.shape[1]), jnp.float32), xb * xb,
                          dn, preferred_element_type=jnp.float32)
    s = s - csq[0:1]
    u = lax.bitcast_convert_type(s, jnp.int32)
    col = lax.broadcasted_iota(jnp.int32, s.shape, 1)
    key = lax.bitcast_convert_type((u & -1024) | (1023 - col), jnp.float32)
    tops = [jnp.max(key, axis=1, keepdims=True)]
    for _ in range(KNN - 1):
        cand = jnp.where(key < tops[-1], key, -jnp.inf)
        tops.append(jnp.max(cand, axis=1, keepdims=True))
    m = lax.bitcast_convert_type(
        jnp.concatenate(tops, axis=1), jnp.int32)
    idx_ref[...] = (1023 - (m & 1023)) + b * N


def _topk(a):
    c = a.shape[1]
    nbat = a.shape[0] // N
    r = 256
    nb = N // r
    return pl.pallas_call(
        _topk_body,
        grid=(nbat, nb),
        in_specs=[
            pl.BlockSpec((N, c), lambda b, rb: (b, 0)),
            pl.BlockSpec((r, c), lambda b, rb: (b * nb + rb, 0)),
        ],
        out_specs=pl.BlockSpec((r, KNN), lambda b, rb: (b * nb + rb, 0)),
        out_shape=jax.ShapeDtypeStruct((nbat * N, KNN), jnp.int32),
    )(a, a)


def _lrelu(z):
    return jnp.where(z >= 0, z, 0.2 * z)


def _mm_uv_body(x_ref, aw_ref, dw_ref, b_ref, u_ref, v_ref):
    x = x_ref[...]
    u_ref[...] = jnp.dot(x, aw_ref[...], preferred_element_type=jnp.float32)
    v_ref[...] = (jnp.dot(x, dw_ref[...], preferred_element_type=jnp.float32)
                  + b_ref[...])


def _mm_uv(a, aw, dw, bias):
    c, co = aw.shape
    cop = max(co, 128)
    if cop != co:
        aw = jnp.concatenate(
            [aw, jnp.zeros((c, cop - co), jnp.float32)], axis=1)
    rows = a.shape[0]
    m = 512
    nb = rows // m
    return pl.pallas_call(
        _mm_uv_body,
        grid=(nb,),
        in_specs=[
            pl.BlockSpec((m, c), lambda i: (i, 0)),
            pl.BlockSpec((c, cop), lambda i: (0, 0)),
            pl.BlockSpec((c, co), lambda i: (0, 0)),
            pl.BlockSpec((1, co), lambda i: (0, 0)),
        ],
        out_specs=[
            pl.BlockSpec((m, cop), lambda i: (i, 0)),
            pl.BlockSpec((m, co), lambda i: (i, 0)),
        ],
        out_shape=[
            jax.ShapeDtypeStruct((rows, cop), jnp.float32),
            jax.ShapeDtypeStruct((rows, co), jnp.float32),
        ],
    )(a, aw, dw, bias.reshape(1, co))


def _mm_body(x_ref, w_ref, b_ref, o_ref, *, act):
    z = jnp.dot(x_ref[...], w_ref[...], preferred_element_type=jnp.float32)
    z = z + b_ref[...]
    o_ref[...] = _lrelu(z) if act else z


def _mm(x, w, bias, act=True, m=2048):
    c, co = w.shape
    rows = x.shape[0]
    nb = rows // m
    return pl.pallas_call(
        functools.partial(_mm_body, act=act),
        grid=(nb,),
        in_specs=[
            pl.BlockSpec((m, c), lambda i: (i, 0)),
            pl.BlockSpec((c, co), lambda i: (0, 0)),
            pl.BlockSpec((1, co), lambda i: (0, 0)),
        ],
        out_specs=pl.BlockSpec((m, co), lambda i: (i, 0)),
        out_shape=jax.ShapeDtypeStruct((rows, co), jnp.float32),
    )(x, w, bias.reshape(1, co))


def _mm_rowadd_body(x_ref, w_ref, t_ref, o_ref):
    z = jnp.dot(x_ref[...], w_ref[...], preferred_element_type=jnp.float32)
    o_ref[...] = _lrelu(z + t_ref[0])


def _mm_rowadd(x, w, t):
    c, co = w.shape
    m = 1024
    nb = (B * N) // m
    per_b = N // m
    return pl.pallas_call(
        _mm_rowadd_body,
        grid=(nb,),
        in_specs=[
            pl.BlockSpec((m, c), lambda i: (i, 0)),
            pl.BlockSpec((c, co), lambda i: (0, 0)),
            pl.BlockSpec((1, 1, co), lambda i: (i // per_b, 0, 0)),
        ],
        out_specs=pl.BlockSpec((m, co), lambda i: (i, 0)),
        out_shape=jax.ShapeDtypeStruct((B * N, co), jnp.float32),
    )(x, w, t)


def _pool_body(h_ref, mx_ref, av_ref):
    h = h_ref[...]
    mx_ref[0] = jnp.max(h, axis=0, keepdims=True)
    av_ref[0] = jnp.sum(h, axis=0, keepdims=True) * (1.0 / N)


def _pool(h):
    e = h.shape[1]
    return pl.pallas_call(
        _pool_body,
        grid=(B,),
        in_specs=[pl.BlockSpec((N, e), lambda b: (b, 0))],
        out_specs=[
            pl.BlockSpec((1, 1, e), lambda b: (b, 0, 0)),
            pl.BlockSpec((1, 1, e), lambda b: (b, 0, 0)),
        ],
        out_shape=[
            jax.ShapeDtypeStruct((B, 1, e), jnp.float32),
            jax.ShapeDtypeStruct((B, 1, e), jnp.float32),
        ],
    )(h)


def _tvec_body(g_ref, w_ref, b_ref, o_ref):
    o_ref[...] = (jnp.dot(g_ref[...], w_ref[...],
                          preferred_element_type=jnp.float32) + b_ref[...])


def _tvec(gcat, w, bias):
    c, co = w.shape
    return pl.pallas_call(
        _tvec_body,
        grid=(1,),
        in_specs=[
            pl.BlockSpec((B, c), lambda i: (0, 0)),
            pl.BlockSpec((c, co), lambda i: (0, 0)),
            pl.BlockSpec((1, co), lambda i: (0, 0)),
        ],
        out_specs=pl.BlockSpec((B, co), lambda i: (0, 0)),
        out_shape=jax.ShapeDtypeStruct((B, co), jnp.float32),
    )(gcat, w, bias.reshape(1, co))


def _mm_lsm_body(x_ref, w_ref, o_ref, *, valid):
    z = jnp.dot(x_ref[...], w_ref[...], preferred_element_type=jnp.float32)
    col = lax.broadcasted_iota(jnp.int32, z.shape, 1)
    ok = col < valid
    zm = jnp.where(ok, z, -jnp.inf)
    mx = jnp.max(zm, axis=1, keepdims=True)
    e = jnp.where(ok, jnp.exp(z - mx), 0.0)
    s = jnp.sum(e, axis=1, keepdims=True)
    o_ref[...] = z - mx - jnp.log(s)


def _mm_lsm(x, w, valid):
    c, co = w.shape
    m = 2048
    nb = (B * N) // m
    return pl.pallas_call(
        functools.partial(_mm_lsm_body, valid=valid),
        grid=(nb,),
        in_specs=[
            pl.BlockSpec((m, c), lambda i: (i, 0)),
            pl.BlockSpec((c, co), lambda i: (0, 0)),
        ],
        out_specs=pl.BlockSpec((m, co), lambda i: (i, 0)),
        out_shape=jax.ShapeDtypeStruct((B * N, co), jnp.float32),
    )(x, w)


def _sc_gather_max(u, v, idx, co):
    pts = u.shape[0]
    cop = u.shape[1]
    per_w = pts // _NW
    ch = 16 if cop <= 128 else 8
    rounds = per_w // ch
    mesh = plsc.VectorSubcoreMesh(
        core_axis_name="c", subcore_axis_name="s",
        num_cores=_NC, num_subcores=_NS)

    @functools.partial(
        pl.kernel,
        out_type=jax.ShapeDtypeStruct((pts, co), jnp.float32),
        mesh=mesh,
        scratch_types=[
            pltpu.VMEM((ch * KNN,), jnp.int32),
            pltpu.VMEM((ch * KNN,), jnp.int32),
            pltpu.VMEM((ch * KNN, cop), jnp.float32),
            pltpu.VMEM((ch * KNN, cop), jnp.float32),
            pltpu.VMEM((ch, co), jnp.float32),
            pltpu.VMEM((ch, co), jnp.float32),
            pltpu.VMEM((ch, co), jnp.float32),
            pltpu.VMEM((ch, co), jnp.float32),
            pltpu.SemaphoreType.DMA,
            pltpu.SemaphoreType.DMA,
            pltpu.SemaphoreType.DMA,
            pltpu.SemaphoreType.DMA,
            pltpu.SemaphoreType.DMA,
            pltpu.SemaphoreType.DMA,
        ],
    )
    def k(u_hbm, v_hbm, idx_hbm, out_hbm,
          idx0, idx1, rows0, rows1, v0, v1, o0, o1,
          gs0, gs1, vs0, vs1, os0, os1):
        wid = lax.axis_index("s") * _NC + lax.axis_index("c")
        base = wid * per_w
        idx_b = (idx0, idx1)
        rows_b = (rows0, rows1)
        v_b = (v0, v1)
        o_b = (o0, o1)
        gsems = (gs0, gs1)
        vsems = (vs0, vs1)
        osems = (os0, os1)

        def issue(r, b):
            p0 = base + r * ch
            pltpu.sync_copy(idx_hbm.at[pl.ds(p0 * KNN, ch * KNN)], idx_b[b])
            pltpu.async_copy(u_hbm.at[idx_b[b]], rows_b[b], gsems[b])
            pltpu.async_copy(v_hbm.at[pl.ds(p0, ch)], v_b[b], vsems[b])

        issue(0, 0)

        def pair_body(i, carry):
            for b in range(2):
                r = 2 * i + b
                nxt = r + 1

                @pl.when(nxt < rounds)
                def _():
                    issue(nxt, 1 - b)

                p0 = base + r * ch

                @pl.when(r >= 2)
                def _():
                    pltpu.make_async_copy(
                        o_b[b], out_hbm.at[pl.ds(p0, ch)], osems[b]).wait()

                pltpu.make_async_copy(
                    u_hbm.at[idx_b[b]], rows_b[b], gsems[b]).wait()
                pltpu.make_async_copy(
                    v_hbm.at[pl.ds(p0, ch)], v_b[b], vsems[b]).wait()
                rows = rows_b[b]
                v_v = v_b[b]
                o_v = o_b[b]

                def pt_body(p, c2):
                    for c in range(co // 16):
                        sl = pl.ds(c * 16, 16)
                        vals = [rows[p * KNN + q, sl] for q in range(KNN)]
                        while len(vals) > 1:
                            nxt_vals = []
                            for j in range(0, len(vals) - 1, 2):
                                nxt_vals.append(
                                    jnp.maximum(vals[j], vals[j + 1]))
                            if len(vals) % 2:
                                nxt_vals.append(vals[-1])
                            vals = nxt_vals
                        z = vals[0] + v_v[p, sl]
                        o_v[p, sl] = jnp.where(z >= 0, z, 0.2 * z)
                    return c2

                lax.fori_loop(0, ch, pt_body, 0)
                pltpu.async_copy(o_v, out_hbm.at[pl.ds(p0, ch)], osems[b])
            return carry

        lax.fori_loop(0, rounds // 2, pair_body, 0)
        for b in range(2):
            r = rounds - 2 + b
            p0 = base + r * ch
            pltpu.make_async_copy(
                o_b[b], out_hbm.at[pl.ds(p0, ch)], osems[b]).wait()

    return k(u, v, idx.reshape(-1))


def _edge_stage(a, w, g, bias):
    c = a.shape[1]
    co = w.shape[0]
    sg = (g / jnp.sqrt(1.0 + 1e-5))[:, None]
    aw = (w[:, :c] * sg).T
    dw = ((w[:, c:] - w[:, :c]) * sg).T
    idx = _topk(a)
    u, v = _mm_uv(a, aw, dw, bias)
    return _sc_gather_max(u, v, idx, co)


def kernel(x, params):
    p = params
    a0 = x.transpose(0, 2, 1).reshape(B * N, 3)

    nchain = 2
    part = (B // nchain) * N
    feats = []
    for h in range(nchain):
        a = a0[h * part:(h + 1) * part]
        x1 = _edge_stage(a, p['W1'], p['g1'], p['b1'])
        x2 = _edge_stage(x1, p['W2'], p['g2'], p['b2'])
        x3 = _edge_stage(x2, p['W3'], p['g3'], p['b3'])
        x4 = _edge_stage(x3, p['W4'], p['g4'], p['b4'])
        feats.append((x1, x2, x3, x4))

    x1, x2, x3, x4 = (
        jnp.concatenate([f[i] for f in feats], axis=0) for i in range(4))
    cat4 = jnp.concatenate([x1, x2, x3, x4], axis=1)
    s5 = (p['g5'] / jnp.sqrt(1.0 + 1e-3))[:, None]
    h5 = _mm(cat4, (p['W5'] * s5).T, p['b5'])

    gmax, gavg = _pool(h5)
    gcat = jnp.concatenate([gmax[:, 0], gavg[:, 0]], axis=1)

    emb2 = gcat.shape[1]
    s106 = (p['g108'] / jnp.sqrt(1.0 + 1e-3))[:, None]
    w106 = p['W106'] * s106
    t = _tvec(gcat, w106[:, :emb2].T, p['b108'])
    h6 = _mm_rowadd(x2, w106[:, emb2:].T, t[:, None, :])

    s107 = (p['g109'] / jnp.sqrt(1.0 + 1e-3))[:, None]
    h7 = _mm(h6, (p['W107'] * s107).T, p['b109'])
    s108 = (p['g1010'] / jnp.sqrt(1.0 + 1e-3))[:, None]
    h8 = _mm(h7, (p['W108'] * s108).T, p['b1010'])

    w109 = jnp.zeros((128, 32), jnp.float32).at[:, :27].set(p['W109'].T)
    out = _mm_lsm(h8, w109, 27)
    return out.reshape(B, N, 32)[:, :, :27]


# speedup vs baseline: 1.9043x; 1.0454x over previous
import functools

import jax
import jax.numpy as jnp
from jax import lax
from jax.experimental import pallas as pl
from jax.experimental.pallas import tpu as pltpu
from jax.experimental.pallas import tpu_sc as plsc

B = 16
N = 1024
KNN = 20
IMIN = -(2**31)

try:
    _info = plsc.get_sparse_core_info()
    _NC, _NS = _info.num_cores, _info.num_subcores
except ValueError:
    _NC, _NS = 2, 16
_NW = _NC * _NS


def _topk_body(xb_ref, xr_ref, idx_ref):
    b = pl.program_id(0)
    xb = xb_ref[...]
    xr = xr_ref[...]
    dn = (((1,), (1,)), ((), ()))
    s = 2.0 * lax.dot_general(xr, xb, dn,
                              preferred_element_type=jnp.float32)
    csq = lax.dot_general(jnp.ones((8, xb.shape[1]), jnp.float32), xb * xb,
                          dn, preferred_element_type=jnp.float32)
    s = s - csq[0:1]
    u = lax.bitcast_convert_type(s, jnp.int32)
    col = lax.broadcasted_iota(jnp.int32, s.shape, 1)
    key = lax.bitcast_convert_type((u & -1024) | (1023 - col), jnp.float32)
    tops = [jnp.max(key, axis=1, keepdims=True)]
    for _ in range(KNN - 1):
        cand = jnp.where(key < tops[-1], key, -jnp.inf)
        tops.append(jnp.max(cand, axis=1, keepdims=True))
    m = lax.bitcast_convert_type(
        jnp.concatenate(tops, axis=1), jnp.int32)
    idx_ref[...] = (1023 - (m & 1023)) + b * N


def _topk(a):
    c = a.shape[1]
    nbat = a.shape[0] // N
    r = 256
    nb = N // r
    return pl.pallas_call(
        _topk_body,
        grid=(nbat, nb),
        in_specs=[
            pl.BlockSpec((N, c), lambda b, rb: (b, 0)),
            pl.BlockSpec((r, c), lambda b, rb: (b * nb + rb, 0)),
        ],
        out_specs=pl.BlockSpec((r, KNN), lambda b, rb: (b * nb + rb, 0)),
        out_shape=jax.ShapeDtypeStruct((nbat * N, KNN), jnp.int32),
    )(a, a)


def _lrelu(z):
    return jnp.where(z >= 0, z, 0.2 * z)


def _mm_uv_body(x_ref, aw_ref, dw_ref, b_ref, u_ref, v_ref):
    x = x_ref[...]
    u_ref[...] = jnp.dot(x, aw_ref[...], preferred_element_type=jnp.float32)
    v_ref[...] = (jnp.dot(x, dw_ref[...], preferred_element_type=jnp.float32)
                  + b_ref[...])


def _mm_uv(a, aw, dw, bias):
    c, co = aw.shape
    cop = max(co, 128)
    if cop != co:
        aw = jnp.concatenate(
            [aw, jnp.zeros((c, cop - co), jnp.float32)], axis=1)
    rows = a.shape[0]
    m = 1024
    nb = rows // m
    return pl.pallas_call(
        _mm_uv_body,
        grid=(nb,),
        in_specs=[
            pl.BlockSpec((m, c), lambda i: (i, 0)),
            pl.BlockSpec((c, cop), lambda i: (0, 0)),
            pl.BlockSpec((c, co), lambda i: (0, 0)),
            pl.BlockSpec((1, co), lambda i: (0, 0)),
        ],
        out_specs=[
            pl.BlockSpec((m, cop), lambda i: (i, 0)),
            pl.BlockSpec((m, co), lambda i: (i, 0)),
        ],
        out_shape=[
            jax.ShapeDtypeStruct((rows, cop), jnp.float32),
            jax.ShapeDtypeStruct((rows, co), jnp.float32),
        ],
    )(a, aw, dw, bias.reshape(1, co))


def _mm_body(x_ref, w_ref, b_ref, o_ref, *, act):
    z = jnp.dot(x_ref[...], w_ref[...], preferred_element_type=jnp.float32)
    z = z + b_ref[...]
    o_ref[...] = _lrelu(z) if act else z


def _mm(x, w, bias, act=True, m=2048):
    c, co = w.shape
    rows = x.shape[0]
    nb = rows // m
    return pl.pallas_call(
        functools.partial(_mm_body, act=act),
        grid=(nb,),
        in_specs=[
            pl.BlockSpec((m, c), lambda i: (i, 0)),
            pl.BlockSpec((c, co), lambda i: (0, 0)),
            pl.BlockSpec((1, co), lambda i: (0, 0)),
        ],
        out_specs=pl.BlockSpec((m, co), lambda i: (i, 0)),
        out_shape=jax.ShapeDtypeStruct((rows, co), jnp.float32),
    )(x, w, bias.reshape(1, co))


def _mm_rowadd_body(x_ref, w_ref, t_ref, o_ref):
    z = jnp.dot(x_ref[...], w_ref[...], preferred_element_type=jnp.float32)
    o_ref[...] = _lrelu(z + t_ref[0])


def _mm_rowadd(x, w, t):
    c, co = w.shape
    m = 1024
    nb = (B * N) // m
    per_b = N // m
    return pl.pallas_call(
        _mm_rowadd_body,
        grid=(nb,),
        in_specs=[
            pl.BlockSpec((m, c), lambda i: (i, 0)),
            pl.BlockSpec((c, co), lambda i: (0, 0)),
            pl.BlockSpec((1, 1, co), lambda i: (i // per_b, 0, 0)),
        ],
        out_specs=pl.BlockSpec((m, co), lambda i: (i, 0)),
        out_shape=jax.ShapeDtypeStruct((B * N, co), jnp.float32),
    )(x, w, t)


def _pool_body(h_ref, mx_ref, av_ref):
    h = h_ref[...]
    mx_ref[0] = jnp.max(h, axis=0, keepdims=True)
    av_ref[0] = jnp.sum(h, axis=0, keepdims=True) * (1.0 / N)


def _pool(h):
    e = h.shape[1]
    nbat = h.shape[0] // N
    return pl.pallas_call(
        _pool_body,
        grid=(nbat,),
        in_specs=[pl.BlockSpec((N, e), lambda b: (b, 0))],
        out_specs=[
            pl.BlockSpec((1, 1, e), lambda b: (b, 0, 0)),
            pl.BlockSpec((1, 1, e), lambda b: (b, 0, 0)),
        ],
        out_shape=[
            jax.ShapeDtypeStruct((nbat, 1, e), jnp.float32),
            jax.ShapeDtypeStruct((nbat, 1, e), jnp.float32),
        ],
    )(h)


def _tvec_body(g_ref, w_ref, b_ref, o_ref):
    o_ref[...] = (jnp.dot(g_ref[...], w_ref[...],
                          preferred_element_type=jnp.float32) + b_ref[...])


def _tvec(gcat, w, bias):
    c, co = w.shape
    return pl.pallas_call(
        _tvec_body,
        grid=(1,),
        in_specs=[
            pl.BlockSpec((B, c), lambda i: (0, 0)),
            pl.BlockSpec((c, co), lambda i: (0, 0)),
            pl.BlockSpec((1, co), lambda i: (0, 0)),
        ],
        out_specs=pl.BlockSpec((B, co), lambda i: (0, 0)),
        out_shape=jax.ShapeDtypeStruct((B, co), jnp.float32),
    )(gcat, w, bias.reshape(1, co))


def _mm_lsm_body(x_ref, w_ref, o_ref, *, valid):
    z = jnp.dot(x_ref[...], w_ref[...], preferred_element_type=jnp.float32)
    col = lax.broadcasted_iota(jnp.int32, z.shape, 1)
    ok = col < valid
    zm = jnp.where(ok, z, -jnp.inf)
    mx = jnp.max(zm, axis=1, keepdims=True)
    e = jnp.where(ok, jnp.exp(z - mx), 0.0)
    s = jnp.sum(e, axis=1, keepdims=True)
    o_ref[...] = z - mx - jnp.log(s)


def _mm_lsm(x, w, valid):
    c, co = w.shape
    m = 2048
    nb = (B * N) // m
    return pl.pallas_call(
        functools.partial(_mm_lsm_body, valid=valid),
        grid=(nb,),
        in_specs=[
            pl.BlockSpec((m, c), lambda i: (i, 0)),
            pl.BlockSpec((c, co), lambda i: (0, 0)),
        ],
        out_specs=pl.BlockSpec((m, co), lambda i: (i, 0)),
        out_shape=jax.ShapeDtypeStruct((B * N, co), jnp.float32),
    )(x, w)


def _sc_gather_max(u, v, idx, co):
    pts = u.shape[0]
    cop = u.shape[1]
    per_w = pts // _NW
    ch = 16 if cop <= 128 else 8
    rounds = per_w // ch
    mesh = plsc.VectorSubcoreMesh(
        core_axis_name="c", subcore_axis_name="s",
        num_cores=_NC, num_subcores=_NS)

    @functools.partial(
        pl.kernel,
        out_type=jax.ShapeDtypeStruct((pts, co), jnp.float32),
        mesh=mesh,
        scratch_types=[
            pltpu.VMEM((ch * KNN,), jnp.int32),
            pltpu.VMEM((ch * KNN,), jnp.int32),
            pltpu.VMEM((ch * KNN, cop), jnp.float32),
            pltpu.VMEM((ch * KNN, cop), jnp.float32),
            pltpu.VMEM((ch, co), jnp.float32),
            pltpu.VMEM((ch, co), jnp.float32),
            pltpu.VMEM((ch, co), jnp.float32),
            pltpu.VMEM((ch, co), jnp.float32),
            pltpu.SemaphoreType.DMA,
            pltpu.SemaphoreType.DMA,
            pltpu.SemaphoreType.DMA,
            pltpu.SemaphoreType.DMA,
            pltpu.SemaphoreType.DMA,
            pltpu.SemaphoreType.DMA,
        ],
    )
    def k(u_hbm, v_hbm, idx_hbm, out_hbm,
          idx0, idx1, rows0, rows1, v0, v1, o0, o1,
          gs0, gs1, vs0, vs1, os0, os1):
        wid = lax.axis_index("s") * _NC + lax.axis_index("c")
        base = wid * per_w
        idx_b = (idx0, idx1)
        rows_b = (rows0, rows1)
        v_b = (v0, v1)
        o_b = (o0, o1)
        gsems = (gs0, gs1)
        vsems = (vs0, vs1)
        osems = (os0, os1)

        def issue(r, b):
            p0 = base + r * ch
            pltpu.sync_copy(idx_hbm.at[pl.ds(p0 * KNN, ch * KNN)], idx_b[b])
            pltpu.async_copy(u_hbm.at[idx_b[b]], rows_b[b], gsems[b])
            pltpu.async_copy(v_hbm.at[pl.ds(p0, ch)], v_b[b], vsems[b])

        issue(0, 0)

        def pair_body(i, carry):
            for b in range(2):
                r = 2 * i + b
                nxt = r + 1

                @pl.when(nxt < rounds)
                def _():
                    issue(nxt, 1 - b)

                p0 = base + r * ch

                @pl.when(r >= 2)
                def _():
                    pltpu.make_async_copy(
                        o_b[b], out_hbm.at[pl.ds(p0, ch)], osems[b]).wait()

                pltpu.make_async_copy(
                    u_hbm.at[idx_b[b]], rows_b[b], gsems[b]).wait()
                pltpu.make_async_copy(
                    v_hbm.at[pl.ds(p0, ch)], v_b[b], vsems[b]).wait()
                rows = rows_b[b]
                v_v = v_b[b]
                o_v = o_b[b]

                def pt_body(p, c2):
                    for c in range(co // 16):
                        sl = pl.ds(c * 16, 16)
                        vals = [rows[p * KNN + q, sl] for q in range(KNN)]
                        while len(vals) > 1:
                            nxt_vals = []
                            for j in range(0, len(vals) - 1, 2):
                                nxt_vals.append(
                                    jnp.maximum(vals[j], vals[j + 1]))
                            if len(vals) % 2:
                                nxt_vals.append(vals[-1])
                            vals = nxt_vals
                        z = vals[0] + v_v[p, sl]
                        o_v[p, sl] = jnp.where(z >= 0, z, 0.2 * z)
                    return c2

                lax.fori_loop(0, ch, pt_body, 0)
                pltpu.async_copy(o_v, out_hbm.at[pl.ds(p0, ch)], osems[b])
            return carry

        lax.fori_loop(0, rounds // 2, pair_body, 0)
        for b in range(2):
            r = rounds - 2 + b
            p0 = base + r * ch
            pltpu.make_async_copy(
                o_b[b], out_hbm.at[pl.ds(p0, ch)], osems[b]).wait()

    return k(u, v, idx.reshape(-1))


def _edge_stage(a, w, g, bias):
    c = a.shape[1]
    co = w.shape[0]
    sg = (g / jnp.sqrt(1.0 + 1e-5))[:, None]
    aw = (w[:, :c] * sg).T
    dw = ((w[:, c:] - w[:, :c]) * sg).T
    idx = _topk(a)
    u, v = _mm_uv(a, aw, dw, bias)
    return _sc_gather_max(u, v, idx, co)


def kernel(x, params):
    p = params
    a0 = x.transpose(0, 2, 1).reshape(B * N, 3)

    nchain = 2
    part = (B // nchain) * N
    s5 = (p['g5'] / jnp.sqrt(1.0 + 1e-3))[:, None]
    w5 = (p['W5'] * s5).T
    feats = []
    pools = []
    for h in range(nchain):
        a = a0[h * part:(h + 1) * part]
        x1 = _edge_stage(a, p['W1'], p['g1'], p['b1'])
        x2 = _edge_stage(x1, p['W2'], p['g2'], p['b2'])
        x3 = _edge_stage(x2, p['W3'], p['g3'], p['b3'])
        x4 = _edge_stage(x3, p['W4'], p['g4'], p['b4'])
        feats.append((x1, x2, x3, x4))
        cat_h = jnp.concatenate([x1, x2, x3, x4], axis=1)
        h5 = _mm(cat_h, w5, p['b5'])
        pools.append(_pool(h5))

    x2 = jnp.concatenate([f[1] for f in feats], axis=0)
    gmax = jnp.concatenate([q[0][:, 0] for q in pools], axis=0)
    gavg = jnp.concatenate([q[1][:, 0] for q in pools], axis=0)
    gcat = jnp.concatenate([gmax, gavg], axis=1)

    emb2 = gcat.shape[1]
    s106 = (p['g108'] / jnp.sqrt(1.0 + 1e-3))[:, None]
    w106 = p['W106'] * s106
    t = _tvec(gcat, w106[:, :emb2].T, p['b108'])
    h6 = _mm_rowadd(x2, w106[:, emb2:].T, t[:, None, :])

    s107 = (p['g109'] / jnp.sqrt(1.0 + 1e-3))[:, None]
    h7 = _mm(h6, (p['W107'] * s107).T, p['b109'])
    s108 = (p['g1010'] / jnp.sqrt(1.0 + 1e-3))[:, None]
    h8 = _mm(h7, (p['W108'] * s108).T, p['b1010'])

    w109 = jnp.zeros((128, 32), jnp.float32).at[:, :27].set(p['W109'].T)
    out = _mm_lsm(h8, w109, 27)
    return out.reshape(B, N, 32)[:, :, :27]


# speedup vs baseline: 1.9045x; 1.0001x over previous
import functools

import jax
import jax.numpy as jnp
from jax import lax
from jax.experimental import pallas as pl
from jax.experimental.pallas import tpu as pltpu
from jax.experimental.pallas import tpu_sc as plsc

B = 16
N = 1024
KNN = 20

try:
    _info = plsc.get_sparse_core_info()
    _NC, _NS = _info.num_cores, _info.num_subcores
except ValueError:
    _NC, _NS = 2, 16
_NW = _NC * _NS


def _topk_body(xb_ref, xr_ref, idx_ref):
    b = pl.program_id(0)
    xb = xb_ref[...]
    xr = xr_ref[...]
    dn = (((1,), (1,)), ((), ()))
    s = 2.0 * lax.dot_general(xr, xb, dn,
                              preferred_element_type=jnp.float32)
    csq = lax.dot_general(jnp.ones((8, xb.shape[1]), jnp.float32), xb * xb,
                          dn, preferred_element_type=jnp.float32)
    s = s - csq[0:1]
    u = lax.bitcast_convert_type(s, jnp.int32)
    col = lax.broadcasted_iota(jnp.int32, s.shape, 1)
    key = lax.bitcast_convert_type((u & -1024) | (1023 - col), jnp.float32)
    tops = [jnp.max(key, axis=1, keepdims=True)]
    for _ in range(KNN - 1):
        cand = jnp.where(key < tops[-1], key, -jnp.inf)
        tops.append(jnp.max(cand, axis=1, keepdims=True))
    m = lax.bitcast_convert_type(
        jnp.concatenate(tops, axis=1), jnp.int32)
    idx_ref[...] = (1023 - (m & 1023)) + b * N


def _topk(a):
    c = a.shape[1]
    nbat = a.shape[0] // N
    r = 256
    nb = N // r
    return pl.pallas_call(
        _topk_body,
        grid=(nbat, nb),
        in_specs=[
            pl.BlockSpec((N, c), lambda b, rb: (b, 0)),
            pl.BlockSpec((r, c), lambda b, rb: (b * nb + rb, 0)),
        ],
        out_specs=pl.BlockSpec((r, KNN), lambda b, rb: (b * nb + rb, 0)),
        out_shape=jax.ShapeDtypeStruct((nbat * N, KNN), jnp.int32),
    )(a, a)


def _lrelu(z):
    return jnp.where(z >= 0, z, 0.2 * z)


def _mm_uv_body(x_ref, aw_ref, dw_ref, b_ref, u_ref, v_ref):
    x = x_ref[...]
    u_ref[...] = jnp.dot(x, aw_ref[...], preferred_element_type=jnp.float32)
    v_ref[...] = (jnp.dot(x, dw_ref[...], preferred_element_type=jnp.float32)
                  + b_ref[...])


def _mm_uv(a, aw, dw, bias):
    c, co = aw.shape
    cop = max(co, 128)
    if cop != co:
        aw = jnp.concatenate(
            [aw, jnp.zeros((c, cop - co), jnp.float32)], axis=1)
    rows = a.shape[0]
    m = 1024
    nb = rows // m
    return pl.pallas_call(
        _mm_uv_body,
        grid=(nb,),
        in_specs=[
            pl.BlockSpec((m, c), lambda i: (i, 0)),
            pl.BlockSpec((c, cop), lambda i: (0, 0)),
            pl.BlockSpec((c, co), lambda i: (0, 0)),
            pl.BlockSpec((1, co), lambda i: (0, 0)),
        ],
        out_specs=[
            pl.BlockSpec((m, cop), lambda i: (i, 0)),
            pl.BlockSpec((m, co), lambda i: (i, 0)),
        ],
        out_shape=[
            jax.ShapeDtypeStruct((rows, cop), jnp.float32),
            jax.ShapeDtypeStruct((rows, co), jnp.float32),
        ],
    )(a, aw, dw, bias.reshape(1, co))


def _mm_body(x_ref, w_ref, b_ref, o_ref, *, act):
    z = jnp.dot(x_ref[...], w_ref[...], preferred_element_type=jnp.float32)
    z = z + b_ref[...]
    o_ref[...] = _lrelu(z) if act else z


def _mm(x, w, bias, act=True, m=2048):
    c, co = w.shape
    rows = x.shape[0]
    nb = rows // m
    return pl.pallas_call(
        functools.partial(_mm_body, act=act),
        grid=(nb,),
        in_specs=[
            pl.BlockSpec((m, c), lambda i: (i, 0)),
            pl.BlockSpec((c, co), lambda i: (0, 0)),
            pl.BlockSpec((1, co), lambda i: (0, 0)),
        ],
        out_specs=pl.BlockSpec((m, co), lambda i: (i, 0)),
        out_shape=jax.ShapeDtypeStruct((rows, co), jnp.float32),
    )(x, w, bias.reshape(1, co))


def _mm_rowadd_body(x_ref, w_ref, t_ref, o_ref):
    z = jnp.dot(x_ref[...], w_ref[...], preferred_element_type=jnp.float32)
    o_ref[...] = _lrelu(z + t_ref[0])


def _mm_rowadd(x, w, t):
    c, co = w.shape
    m = 1024
    nb = (B * N) // m
    per_b = N // m
    return pl.pallas_call(
        _mm_rowadd_body,
        grid=(nb,),
        in_specs=[
            pl.BlockSpec((m, c), lambda i: (i, 0)),
            pl.BlockSpec((c, co), lambda i: (0, 0)),
            pl.BlockSpec((1, 1, co), lambda i: (i // per_b, 0, 0)),
        ],
        out_specs=pl.BlockSpec((m, co), lambda i: (i, 0)),
        out_shape=jax.ShapeDtypeStruct((B * N, co), jnp.float32),
    )(x, w, t)


def _pool_body(h_ref, mx_ref, av_ref):
    h = h_ref[...]
    mx_ref[0] = jnp.max(h, axis=0, keepdims=True)
    av_ref[0] = jnp.sum(h, axis=0, keepdims=True) * (1.0 / N)


def _pool(h):
    e = h.shape[1]
    nbat = h.shape[0] // N
    return pl.pallas_call(
        _pool_body,
        grid=(nbat,),
        in_specs=[pl.BlockSpec((N, e), lambda b: (b, 0))],
        out_specs=[
            pl.BlockSpec((1, 1, e), lambda b: (b, 0, 0)),
            pl.BlockSpec((1, 1, e), lambda b: (b, 0, 0)),
        ],
        out_shape=[
            jax.ShapeDtypeStruct((nbat, 1, e), jnp.float32),
            jax.ShapeDtypeStruct((nbat, 1, e), jnp.float32),
        ],
    )(h)


def _tvec_body(g_ref, w_ref, b_ref, o_ref):
    o_ref[...] = (jnp.dot(g_ref[...], w_ref[...],
                          preferred_element_type=jnp.float32) + b_ref[...])


def _tvec(gcat, w, bias):
    c, co = w.shape
    return pl.pallas_call(
        _tvec_body,
        grid=(1,),
        in_specs=[
            pl.BlockSpec((B, c), lambda i: (0, 0)),
            pl.BlockSpec((c, co), lambda i: (0, 0)),
            pl.BlockSpec((1, co), lambda i: (0, 0)),
        ],
        out_specs=pl.BlockSpec((B, co), lambda i: (0, 0)),
        out_shape=jax.ShapeDtypeStruct((B, co), jnp.float32),
    )(gcat, w, bias.reshape(1, co))


def _mm_lsm_body(x_ref, w_ref, o_ref, *, valid):
    z = jnp.dot(x_ref[...], w_ref[...], preferred_element_type=jnp.float32)
    col = lax.broadcasted_iota(jnp.int32, z.shape, 1)
    ok = col < valid
    zm = jnp.where(ok, z, -jnp.inf)
    mx = jnp.max(zm, axis=1, keepdims=True)
    e = jnp.where(ok, jnp.exp(z - mx), 0.0)
    s = jnp.sum(e, axis=1, keepdims=True)
    o_ref[...] = z - mx - jnp.log(s)


def _mm_lsm(x, w, valid):
    c, co = w.shape
    m = 2048
    nb = (B * N) // m
    return pl.pallas_call(
        functools.partial(_mm_lsm_body, valid=valid),
        grid=(nb,),
        in_specs=[
            pl.BlockSpec((m, c), lambda i: (i, 0)),
            pl.BlockSpec((c, co), lambda i: (0, 0)),
        ],
        out_specs=pl.BlockSpec((m, co), lambda i: (i, 0)),
        out_shape=jax.ShapeDtypeStruct((B * N, co), jnp.float32),
    )(x, w)


def _sc_gather_max(u, v, idx, co):
    pts = u.shape[0]
    cop = u.shape[1]
    per_w = pts // _NW
    ch = 16 if cop <= 128 else 8
    rounds = per_w // ch
    mesh = plsc.VectorSubcoreMesh(
        core_axis_name="c", subcore_axis_name="s",
        num_cores=_NC, num_subcores=_NS)

    @functools.partial(
        pl.kernel,
        out_type=jax.ShapeDtypeStruct((pts, co), jnp.float32),
        mesh=mesh,
        scratch_types=[
            pltpu.VMEM((ch * KNN,), jnp.int32),
            pltpu.VMEM((ch * KNN,), jnp.int32),
            pltpu.VMEM((ch * KNN, cop), jnp.float32),
            pltpu.VMEM((ch * KNN, cop), jnp.float32),
            pltpu.VMEM((ch, co), jnp.float32),
            pltpu.VMEM((ch, co), jnp.float32),
            pltpu.VMEM((ch, co), jnp.float32),
            pltpu.VMEM((ch, co), jnp.float32),
            pltpu.SemaphoreType.DMA,
            pltpu.SemaphoreType.DMA,
            pltpu.SemaphoreType.DMA,
            pltpu.SemaphoreType.DMA,
            pltpu.SemaphoreType.DMA,
            pltpu.SemaphoreType.DMA,
        ],
    )
    def k(u_hbm, v_hbm, idx_hbm, out_hbm,
          idx0, idx1, rows0, rows1, v0, v1, o0, o1,
          gs0, gs1, vs0, vs1, os0, os1):
        wid = lax.axis_index("s") * _NC + lax.axis_index("c")
        base = wid * per_w
        idx_b = (idx0, idx1)
        rows_b = (rows0, rows1)
        v_b = (v0, v1)
        o_b = (o0, o1)
        gsems = (gs0, gs1)
        vsems = (vs0, vs1)
        osems = (os0, os1)

        def issue(r, b):
            p0 = base + r * ch
            pltpu.sync_copy(idx_hbm.at[pl.ds(p0 * KNN, ch * KNN)], idx_b[b])
            pltpu.async_copy(u_hbm.at[idx_b[b]], rows_b[b], gsems[b])
            pltpu.async_copy(v_hbm.at[pl.ds(p0, ch)], v_b[b], vsems[b])

        issue(0, 0)

        def pair_body(i, carry):
            for b in range(2):
                r = 2 * i + b
                nxt = r + 1

                @pl.when(nxt < rounds)
                def _():
                    issue(nxt, 1 - b)

                p0 = base + r * ch

                @pl.when(r >= 2)
                def _():
                    pltpu.make_async_copy(
                        o_b[b], out_hbm.at[pl.ds(p0, ch)], osems[b]).wait()

                pltpu.make_async_copy(
                    u_hbm.at[idx_b[b]], rows_b[b], gsems[b]).wait()
                pltpu.make_async_copy(
                    v_hbm.at[pl.ds(p0, ch)], v_b[b], vsems[b]).wait()
                rows = rows_b[b]
                v_v = v_b[b]
                o_v = o_b[b]

                def pt_body(p, c2):
                    for c in range(co // 16):
                        sl = pl.ds(c * 16, 16)
                        vals = [rows[p * KNN + q, sl] for q in range(KNN)]
                        while len(vals) > 1:
                            nxt_vals = []
                            for j in range(0, len(vals) - 1, 2):
                                nxt_vals.append(
                                    jnp.maximum(vals[j], vals[j + 1]))
                            if len(vals) % 2:
                                nxt_vals.append(vals[-1])
                            vals = nxt_vals
                        z = vals[0] + v_v[p, sl]
                        o_v[p, sl] = jnp.where(z >= 0, z, 0.2 * z)
                    return c2

                lax.fori_loop(0, ch, pt_body, 0)
                pltpu.async_copy(o_v, out_hbm.at[pl.ds(p0, ch)], osems[b])
            return carry

        lax.fori_loop(0, rounds // 2, pair_body, 0)
        for b in range(2):
            r = rounds - 2 + b
            p0 = base + r * ch
            pltpu.make_async_copy(
                o_b[b], out_hbm.at[pl.ds(p0, ch)], osems[b]).wait()

    return k(u, v, idx.reshape(-1))


def _edge_stage(a, w, g, bias):
    c = a.shape[1]
    co = w.shape[0]
    sg = (g / jnp.sqrt(1.0 + 1e-5))[:, None]
    aw = (w[:, :c] * sg).T
    dw = ((w[:, c:] - w[:, :c]) * sg).T
    idx = _topk(a)
    u, v = _mm_uv(a, aw, dw, bias)
    return _sc_gather_max(u, v, idx, co)


def kernel(x, params):
    p = params
    a0 = x.transpose(0, 2, 1).reshape(B * N, 3)

    nchain = 2
    part = (B // nchain) * N
    s5 = (p['g5'] / jnp.sqrt(1.0 + 1e-3))[:, None]
    w5 = (p['W5'] * s5).T
    feats = []
    pools = []
    for h in range(nchain):
        a = a0[h * part:(h + 1) * part]
        x1 = _edge_stage(a, p['W1'], p['g1'], p['b1'])
        x2 = _edge_stage(x1, p['W2'], p['g2'], p['b2'])
        x3 = _edge_stage(x2, p['W3'], p['g3'], p['b3'])
        x4 = _edge_stage(x3, p['W4'], p['g4'], p['b4'])
        feats.append((x1, x2, x3, x4))
        cat_h = jnp.concatenate([x1, x2, x3, x4], axis=1)
        h5 = _mm(cat_h, w5, p['b5'])
        pools.append(_pool(h5))

    x2 = jnp.concatenate([f[1] for f in feats], axis=0)
    gmax = jnp.concatenate([q[0][:, 0] for q in pools], axis=0)
    gavg = jnp.concatenate([q[1][:, 0] for q in pools], axis=0)
    gcat = jnp.concatenate([gmax, gavg], axis=1)

    emb2 = gcat.shape[1]
    s106 = (p['g108'] / jnp.sqrt(1.0 + 1e-3))[:, None]
    w106 = p['W106'] * s106
    t = _tvec(gcat, w106[:, :emb2].T, p['b108'])
    h6 = _mm_rowadd(x2, w106[:, emb2:].T, t[:, None, :])

    s107 = (p['g109'] / jnp.sqrt(1.0 + 1e-3))[:, None]
    h7 = _mm(h6, (p['W107'] * s107).T, p['b109'])
    s108 = (p['g1010'] / jnp.sqrt(1.0 + 1e-3))[:, None]
    h8 = _mm(h7, (p['W108'] * s108).T, p['b1010'])

    w109 = jnp.zeros((128, 32), jnp.float32).at[:, :27].set(p['W109'].T)
    out = _mm_lsm(h8, w109, 27)
    return out.reshape(B, N, 32)[:, :, :27]
